# Initial kernel scaffold; baseline (speedup 1.0000x reference)
import jax
import jax.numpy as jnp
from jax.experimental import pallas as pl


def kernel(x, edge_index, batch, conv1_Wl, conv1_bl, conv1_Wr, pool1_Wrel, pool1_brel, pool1_Wroot, conv2_Wl, conv2_bl, conv2_Wr, pool2_Wrel, pool2_brel, pool2_Wroot, lin_W, lin_b):
    raise NotImplementedError("write your pallas kernel here")


# speedup vs baseline: 7.0157x; 7.0157x over previous
import functools
import math

import jax
import jax.numpy as jnp
from jax import lax
from jax.experimental import pallas as pl
from jax.experimental.pallas import tpu as pltpu
from jax.experimental.pallas import tpu_sc as plsc

NC = 2
NS = 16
NW = NC * NS
L = 16
CHUNK = 128

MININT = -2147483648


def _mesh():
    return plsc.VectorSubcoreMesh(
        core_axis_name="c", subcore_axis_name="s", num_cores=NC, num_subcores=NS
    )


@functools.lru_cache(maxsize=None)
def _make_segrow(npad, nchunks, d):
    rpt = npad // NS

    @functools.partial(
        pl.kernel,
        out_type=(
            jax.ShapeDtypeStruct((NC, npad, d), jnp.float32),
            jax.ShapeDtypeStruct((NW, npad), jnp.float32),
        ),
        mesh=_mesh(),
        scratch_types=[
            pltpu.VMEM((4, 2, CHUNK), jnp.int32),
            pltpu.VMEM((CHUNK, d), jnp.float32),
            pltpu.VMEM((CHUNK, d), jnp.float32),
            pltpu.VMEM((npad,), jnp.float32),
            pltpu.VMEM_SHARED((npad, d), jnp.float32),
            pltpu.SemaphoreType.DMA,
            pltpu.SemaphoreType.DMA,
            pltpu.SemaphoreType.DMA,
            pltpu.SemaphoreType.DMA,
            pltpu.SemaphoreType.DMA,
            pltpu.SemaphoreType.DMA,
        ],
        compiler_params=pltpu.CompilerParams(needs_layout_passes=False),
    )
    def seg(table, eidx, zrows, agg_out, cnt_out, islot, rb0, rb1, histv,
            aggsh, is0, is1, is2, is3, sm0, sm1):
        c = lax.axis_index("c")
        s = lax.axis_index("s")
        wid = s * NC + c
        pltpu.sync_copy(zrows.at[pl.ds(s * rpt, rpt)],
                        aggsh.at[pl.ds(s * rpt, rpt)])
        isems = (is0, is1, is2, is3)
        rbs = (rb0, rb1)
        sms = (sm0, sm1)

        @pl.loop(0, npad // L)
        def _z(i):
            histv[pl.ds(i * L, L)] = jnp.zeros((L,), jnp.float32)

        for t in range(4):
            pltpu.async_copy(eidx.at[wid, t], islot.at[t], isems[t])
        plsc.subcore_barrier()
        for t in range(2):
            pltpu.make_async_copy(eidx.at[wid, t], islot.at[t], isems[t]).wait()
            pltpu.async_copy(table.at[islot.at[t, 0]], rbs[t], sms[t])

        @pl.loop(0, nchunks, step=4)
        def _r(g):
            for b in range(4):
                j = g + b
                bb = b % 2
                ns = (b + 2) % 4
                pltpu.make_async_copy(table.at[islot.at[b, 0]],
                                      rbs[bb], sms[bb]).wait()
                pltpu.sync_copy(rbs[bb], aggsh.at[islot.at[b, 1]], add=True)
                ones = jnp.ones((L,), jnp.float32)
                for u in range(CHUNK // L):
                    di = islot[b, 1, pl.ds(u * L, L)]
                    plsc.addupdate_scatter(histv, [di], ones)
                pltpu.async_copy(eidx.at[wid, j + 4], islot.at[b], isems[b])
                pltpu.make_async_copy(eidx.at[wid, j + 2], islot.at[ns],
                                      isems[ns]).wait()
                pltpu.async_copy(table.at[islot.at[ns, 0]], rbs[bb], sms[bb])

        for b in range(2):
            pltpu.make_async_copy(table.at[islot.at[b, 0]], rbs[b],
                                  sms[b]).wait()
            pltpu.sync_copy(rbs[b], aggsh.at[islot.at[b, 1]], add=True)
        for t in (2, 3):
            pltpu.make_async_copy(eidx.at[wid, 0], islot.at[t],
                                  isems[t]).wait()

        plsc.subcore_barrier()
        pltpu.sync_copy(aggsh.at[pl.ds(s * rpt, rpt)],
                        agg_out.at[c, pl.ds(s * rpt, rpt)])
        pltpu.sync_copy(histv, cnt_out.at[wid])

    return seg


@functools.lru_cache(maxsize=None)
def _make_segscalar(npad, nchunks):
    @functools.partial(
        pl.kernel,
        out_type=jax.ShapeDtypeStruct((NW, npad), jnp.float32),
        mesh=_mesh(),
        scratch_types=[
            pltpu.VMEM((nchunks, 2, CHUNK), jnp.int32),
            pltpu.VMEM((npad,), jnp.float32),
            pltpu.VMEM((npad,), jnp.float32),
        ],
        compiler_params=pltpu.CompilerParams(needs_layout_passes=False),
    )
    def seg(p, eidx, out, ev, pv, histv):
        c = lax.axis_index("c")
        s = lax.axis_index("s")
        wid = s * NC + c
        pltpu.sync_copy(eidx.at[wid, pl.ds(0, nchunks)], ev)
        pltpu.sync_copy(p, pv)

        @pl.loop(0, npad // L)
        def _z(i):
            histv[pl.ds(i * L, L)] = jnp.zeros((L,), jnp.float32)

        @pl.loop(0, nchunks)
        def _h(j):
            for u in range(CHUNK // L):
                si = ev[j, 0, pl.ds(u * L, L)]
                di = ev[j, 1, pl.ds(u * L, L)]
                vals = plsc.load_gather(pv, [si])
                plsc.addupdate_scatter(histv, [di], vals)

        pltpu.sync_copy(histv, out.at[wid])

    return seg


def _f2key(s):
    b = lax.bitcast_convert_type(s, jnp.int32)
    return jnp.where(b < 0, jnp.bitwise_or(jnp.bitwise_not(b), jnp.int32(MININT)), b)


def _topk_mask(keys2d, k):
    R = keys2d.shape[0]

    def bit_body(t, lo):
        cand = lo + lax.shift_left(jnp.int32(1), jnp.int32(30) - t)
        cnt = jnp.sum((keys2d >= cand).astype(jnp.int32))
        return jnp.where(cnt >= k, cand, lo)

    cnt0 = jnp.sum((keys2d >= 0).astype(jnp.int32))
    lo0 = jnp.where(cnt0 >= k, jnp.int32(0), jnp.int32(MININT))
    T = lax.fori_loop(0, 31, bit_body, lo0)
    gt = keys2d > T
    cnt_gt = jnp.sum(gt.astype(jnp.float32))
    tie = (keys2d == T).astype(jnp.float32)
    ii = lax.broadcasted_iota(jnp.int32, (128, 128), 0)
    jj = lax.broadcasted_iota(jnp.int32, (128, 128), 1)
    upper = (ii <= jj).astype(jnp.float32)
    intra = lax.dot_general(tie, upper, (((1,), (0,)), ((), ())),
                            preferred_element_type=jnp.float32)
    rowtot = intra[:, 127:128]
    ri = lax.broadcasted_iota(jnp.int32, (R, R), 0)
    rj = lax.broadcasted_iota(jnp.int32, (R, R), 1)
    lower = (rj < ri).astype(jnp.float32)
    offs = lax.dot_general(lower, rowtot, (((1,), (0,)), ((), ())),
                           preferred_element_type=jnp.float32)
    tie_rank = intra + offs
    sel_tie = (tie > 0.0) & (tie_rank <= (k - cnt_gt))
    return gt | sel_tie


def _mm(a, b_t):
    return lax.dot_general(a, b_t, (((1,), (1,)), ((), ())),
                           preferred_element_type=jnp.float32,
                           precision=lax.Precision.HIGHEST)


@functools.lru_cache(maxsize=None)
def _make_conv_tc(npad, d):
    R = npad // 128

    def body(aggp, degp, xin, wl, bl, wr, wrel, wroot, pm, h_ref, p_ref, r_ref):
        agg = aggp[0] + aggp[1]
        deg = jnp.sum(degp[...], axis=0)
        invd = 1.0 / jnp.maximum(deg, 1.0)
        invT = jnp.transpose(invd.reshape(R, 128))
        h0 = _mm(agg, wl[...])
        x1 = _mm(xin[...], wr[...])
        blv = bl[...][None, :]
        pm2 = pm[...]
        pmT = jnp.transpose(pm2)
        for j in range(R):
            sl = slice(j * 128, (j + 1) * 128)
            hj = jnp.maximum(h0[sl, :] * invT[:, j:j + 1] + blv + x1[sl, :], 0.0)
            h_ref[sl, :] = hj
            pj = _mm(hj, wrel[...]) * pmT[:, j:j + 1]
            rj = _mm(hj, wroot[...])
            p_ref[j, :] = jnp.transpose(pj)[0, :]
            r_ref[j, :] = jnp.transpose(rj)[0, :]

    return pl.pallas_call(
        body,
        out_shape=(
            jax.ShapeDtypeStruct((npad, d), jnp.float32),
            jax.ShapeDtypeStruct((R, 128), jnp.float32),
            jax.ShapeDtypeStruct((R, 128), jnp.float32),
        ),
    )


@functools.lru_cache(maxsize=None)
def _make_pool_tc(npad, d, n, k):
    R = npad // 128

    def body(sparts, r_in, brel, h_in, selm, hp_ref, m_ref):
        s = jnp.sum(sparts[...], axis=0).reshape(R, 128)
        score = s + brel[0] + r_in[...]
        keys = _f2key(score)
        vi = lax.broadcasted_iota(jnp.int32, (R, 128), 0) * 128 + \
            lax.broadcasted_iota(jnp.int32, (R, 128), 1)
        valid = (vi < n) & (selm[...] > 0.0)
        keys = jnp.where(valid, keys, jnp.int32(MININT))
        m = _topk_mask(keys, k)
        mf = m.astype(jnp.float32)
        m_ref[...] = mf
        scale = jnp.tanh(score) * mf
        scaleT = jnp.transpose(scale)
        for j in range(R):
            sl = slice(j * 128, (j + 1) * 128)
            hp_ref[sl, :] = h_in[sl, :] * scaleT[:, j:j + 1]

    return pl.pallas_call(
        body,
        out_shape=(
            jax.ShapeDtypeStruct((npad, d), jnp.float32),
            jax.ShapeDtypeStruct((R, 128), jnp.float32),
        ),
    )


@functools.lru_cache(maxsize=None)
def _make_final_tc(npad, d, n, k, dout):
    R = npad // 128

    def body(sparts, r_in, brel, h_in, selm, lw, lb, out_ref):
        s = jnp.sum(sparts[...], axis=0).reshape(R, 128)
        score = s + brel[0] + r_in[...]
        keys = _f2key(score)
        vi = lax.broadcasted_iota(jnp.int32, (R, 128), 0) * 128 + \
            lax.broadcasted_iota(jnp.int32, (R, 128), 1)
        valid = (vi < n) & (selm[...] > 0.0)
        keys = jnp.where(valid, keys, jnp.int32(MININT))
        m = _topk_mask(keys, k)
        scale = jnp.tanh(score) * m.astype(jnp.float32)
        scaleT = jnp.transpose(scale)
        acc = jnp.zeros((1, d), jnp.float32)
        for j in range(R):
            sl = slice(j * 128, (j + 1) * 128)
            acc = acc + jnp.sum(h_in[sl, :] * scaleT[:, j:j + 1], axis=0,
                                keepdims=True)
        pooled = acc * (1.0 / k)
        out_ref[...] = _mm(pooled, lw[...]) + lb[...][None, :]

    return pl.pallas_call(
        body,
        out_shape=jax.ShapeDtypeStruct((1, dout), jnp.float32),
    )


def kernel(x, edge_index, batch, conv1_Wl, conv1_bl, conv1_Wr, pool1_Wrel,
           pool1_brel, pool1_Wroot, conv2_Wl, conv2_bl, conv2_Wr, pool2_Wrel,
           pool2_brel, pool2_Wroot, lin_W, lin_b):
    n, d = x.shape
    e = edge_index.shape[1]
    dout = lin_W.shape[0]
    k1 = int(math.ceil(0.5 * n))
    k2 = int(math.ceil(0.5 * k1))
    npad = ((n + 1 + 2047) // 2048) * 2048
    R = npad // 128
    epw = (e + NW - 1) // NW
    nchunks = (epw + CHUNK - 1) // CHUNK
    epad = NW * nchunks * CHUNK

    src = edge_index[0].astype(jnp.int32)
    dst = edge_index[1].astype(jnp.int32)
    srci = jnp.concatenate([src, jnp.zeros((epad - e,), jnp.int32)]
                           ).reshape(NW, nchunks, 1, CHUNK)
    dsti = jnp.concatenate([dst, jnp.full((epad - e,), n, jnp.int32)]
                           ).reshape(NW, nchunks, 1, CHUNK)
    eidx = jnp.concatenate([srci, dsti], axis=2)
    pad4 = jnp.concatenate(
        [jnp.zeros((NW, 4, 1, CHUNK), jnp.int32),
         jnp.full((NW, 4, 1, CHUNK), n, jnp.int32)], axis=2)
    eidxp = jnp.concatenate([eidx, pad4], axis=1)

    xp = jnp.concatenate([x, jnp.zeros((npad - n, d), x.dtype)], axis=0)
    zrows = jnp.zeros((npad, d), jnp.float32)

    segrow = _make_segrow(npad, nchunks, d)
    segsca = _make_segscalar(npad, nchunks)
    conv_tc = _make_conv_tc(npad, d)
    pool_tc = _make_pool_tc(npad, d, n, k1)
    final_tc = _make_final_tc(npad, d, n, k2, dout)

    onesm = jnp.ones((R, 128), jnp.float32)

    agg1, deg1 = segrow(xp, eidxp, zrows)
    h, p1, r1 = conv_tc(agg1, deg1, xp, conv1_Wl, conv1_bl, conv1_Wr,
                        pool1_Wrel, pool1_Wroot, onesm)
    s1 = segsca(p1.reshape(npad), eidxp)
    hp, m1 = pool_tc(s1, r1, pool1_brel, h, onesm)
    agg2, _cnt2 = segrow(hp, eidxp, zrows)
    deg2 = segsca(m1.reshape(npad), eidxp)
    h2, p2, r2 = conv_tc(agg2, deg2, hp, conv2_Wl, conv2_bl, conv2_Wr,
                         pool2_Wrel, pool2_Wroot, m1)
    s2 = segsca(p2.reshape(npad), eidxp)
    return final_tc(s2, r2, pool2_brel, h2, m1, lin_W, lin_b)
